# Initial kernel scaffold; baseline (speedup 1.0000x reference)
import jax
import jax.numpy as jnp
from jax.experimental import pallas as pl


def kernel(x, edge_index, W_self0, W_neigh0, b0, W_self1, W_neigh1, b1):
    raise NotImplementedError("write your pallas kernel here")


# speedup vs baseline: 5.6281x; 5.6281x over previous
import functools

import jax
import jax.numpy as jnp
from jax import lax
from jax.experimental import pallas as pl
from jax.experimental.pallas import tpu as pltpu
from jax.experimental.pallas import tpu_sc as plsc

N_NODES = 10000
N_EDGES = 320000
D_IN = 128
D_HID = 128
D_OUT = 64

NC = 2
NS = 16
NW = NC * NS
N_PAD = 10240
RPT = N_PAD // NS
EPW = N_EDGES // NW
CHUNK = 80
NCHUNK = EPW // CHUNK


def _seg_sum_kernel(d_feat, with_deg):
    mesh = plsc.VectorSubcoreMesh(core_axis_name="c", subcore_axis_name="s",
                                  num_cores=NC, num_subcores=NS)
    out_type = [jax.ShapeDtypeStruct((NC, N_PAD, d_feat), jnp.float32)]
    scratch = [
        pltpu.VMEM_SHARED((N_PAD, d_feat), jnp.float32),
        pltpu.VMEM((CHUNK,), jnp.int32),
        pltpu.VMEM((CHUNK,), jnp.int32),
        pltpu.VMEM((CHUNK, d_feat), jnp.float32),
        pltpu.SemaphoreType.DMA,
    ]
    if with_deg:
        out_type.append(jax.ShapeDtypeStruct((NC, N_PAD, 16), jnp.float32))
        scratch += [
            pltpu.VMEM_SHARED((N_PAD, 16), jnp.float32),
            pltpu.VMEM((CHUNK, 16), jnp.float32),
        ]

    def body(table, src, dst, z_feat, *rest):
        if with_deg:
            (z_deg, acc_out, deg_out, acc_sh, srcv, dstv, rows, sem,
             deg_sh, onesv) = rest
        else:
            (acc_out, acc_sh, srcv, dstv, rows, sem) = rest
        cid = lax.axis_index("c")
        sid = lax.axis_index("s")
        wid = sid * NC + cid
        r0 = sid * RPT

        pltpu.sync_copy(z_feat.at[pl.ds(r0, RPT)], acc_sh.at[pl.ds(r0, RPT)])
        if with_deg:
            pltpu.sync_copy(z_deg.at[pl.ds(r0, RPT)], deg_sh.at[pl.ds(r0, RPT)])
            one = jnp.full((16,), 1.0, dtype=jnp.float32)
            for i in range(CHUNK):
                onesv[i] = one
        plsc.subcore_barrier()

        ebase = wid * EPW

        def step(k, _):
            base = ebase + k * CHUNK
            pltpu.sync_copy(src.at[pl.ds(base, CHUNK)], srcv)
            pltpu.sync_copy(dst.at[pl.ds(base, CHUNK)], dstv)
            pltpu.async_copy(table.at[srcv], rows, sem).wait()
            pltpu.sync_copy(rows, acc_sh.at[dstv], add=True)
            if with_deg:
                pltpu.sync_copy(onesv, deg_sh.at[dstv], add=True)
            return 0

        lax.fori_loop(0, NCHUNK, step, 0)
        plsc.subcore_barrier()

        pltpu.sync_copy(acc_sh.at[pl.ds(r0, RPT)],
                        acc_out.at[cid, pl.ds(r0, RPT)])
        if with_deg:
            pltpu.sync_copy(deg_sh.at[pl.ds(r0, RPT)],
                            deg_out.at[cid, pl.ds(r0, RPT)])

    return pl.kernel(body, out_type=out_type, mesh=mesh,
                     scratch_types=scratch,
                     compiler_params=pltpu.CompilerParams(
                         use_tc_tiling_on_sc=False),
                     name=f"sage_seg_sum_d{d_feat}")


_seg_sum_l0 = _seg_sum_kernel(D_IN, with_deg=True)
_seg_sum_l1 = _seg_sum_kernel(D_OUT, with_deg=False)

_BLK = 400


def _tc_layer_body(x_ref, accp_ref, accd_ref, ws0_ref, wn0_ref, b0_ref,
                   ws1_ref, wn1_ref, b1_ref, p1_ref, s1e_ref):
    sum_p = accp_ref[0] + accp_ref[1]
    deg = accd_ref[0, :, 0:1] + accd_ref[1, :, 0:1]
    d = jnp.maximum(deg, 1.0)
    agg0 = sum_p / d
    x = x_ref[...]
    h = x @ ws0_ref[...] + agg0 @ wn0_ref[...] + b0_ref[...]
    h = jnp.maximum(h, 0.0)
    p1_ref[...] = jnp.dot(h, wn1_ref[...],
                          preferred_element_type=jnp.float32)
    s1 = jnp.dot(h, ws1_ref[...],
                 preferred_element_type=jnp.float32) + b1_ref[...]
    s1e_ref[...] = jnp.concatenate(
        [s1, jnp.broadcast_to(deg, (s1.shape[0], D_OUT))], axis=1)


def _tc_final_body(s1e_ref, accq_ref, out_ref):
    deg = jnp.maximum(s1e_ref[:, D_OUT:D_OUT + 1], 1.0)
    agg1 = (accq_ref[0] + accq_ref[1]) / deg
    out_ref[...] = s1e_ref[:, :D_OUT] + agg1


def kernel(x, edge_index, W_self0, W_neigh0, b0, W_self1, W_neigh1, b1):
    src = edge_index[0]
    dst = edge_index[1]
    z128 = jnp.zeros((N_PAD, D_IN), jnp.float32)
    z64 = jnp.zeros((N_PAD, D_OUT), jnp.float32)
    z16 = jnp.zeros((N_PAD, 16), jnp.float32)

    accp, accd = _seg_sum_l0(x, src, dst, z128, z16)

    nblk = N_NODES // _BLK
    p1, s1e = pl.pallas_call(
        _tc_layer_body,
        grid=(nblk,),
        in_specs=[
            pl.BlockSpec((_BLK, D_IN), lambda i: (i, 0)),
            pl.BlockSpec((NC, _BLK, D_IN), lambda i: (0, i, 0)),
            pl.BlockSpec((NC, _BLK, 16), lambda i: (0, i, 0)),
            pl.BlockSpec((D_IN, D_HID), lambda i: (0, 0)),
            pl.BlockSpec((D_IN, D_HID), lambda i: (0, 0)),
            pl.BlockSpec((1, D_HID), lambda i: (0, 0)),
            pl.BlockSpec((D_HID, D_OUT), lambda i: (0, 0)),
            pl.BlockSpec((D_HID, D_OUT), lambda i: (0, 0)),
            pl.BlockSpec((1, D_OUT), lambda i: (0, 0)),
        ],
        out_specs=[
            pl.BlockSpec((_BLK, D_OUT), lambda i: (i, 0)),
            pl.BlockSpec((_BLK, 2 * D_OUT), lambda i: (i, 0)),
        ],
        out_shape=[
            jax.ShapeDtypeStruct((N_NODES, D_OUT), jnp.float32),
            jax.ShapeDtypeStruct((N_NODES, 2 * D_OUT), jnp.float32),
        ],
        name="sage_tc_layer",
    )(x, accp, accd, W_self0, W_neigh0, b0.reshape(1, D_HID),
      W_self1, W_neigh1, b1.reshape(1, D_OUT))

    (accq,) = _seg_sum_l1(p1, src, dst, z64)

    out = pl.pallas_call(
        _tc_final_body,
        grid=(nblk,),
        in_specs=[
            pl.BlockSpec((_BLK, 2 * D_OUT), lambda i: (i, 0)),
            pl.BlockSpec((NC, _BLK, D_OUT), lambda i: (0, i, 0)),
        ],
        out_specs=pl.BlockSpec((_BLK, D_OUT), lambda i: (i, 0)),
        out_shape=jax.ShapeDtypeStruct((N_NODES, D_OUT), jnp.float32),
        name="sage_tc_final",
    )(s1e, accq)
    return out


# speedup vs baseline: 10.6664x; 1.8952x over previous
import functools

import jax
import jax.numpy as jnp
from jax import lax
from jax.experimental import pallas as pl
from jax.experimental.pallas import tpu as pltpu
from jax.experimental.pallas import tpu_sc as plsc

N_NODES = 10000
N_EDGES = 320000
D_IN = 128
D_HID = 128
D_OUT = 64

NC = 2
NS = 16
NW = NC * NS
N_PAD = 10240
RPT = N_PAD // NS
EPW = N_EDGES // NW
CHUNK = 100
NCHUNK = EPW // CHUNK


def _seg_sum_kernel(d_feat, with_deg):
    mesh = plsc.VectorSubcoreMesh(core_axis_name="c", subcore_axis_name="s",
                                  num_cores=NC, num_subcores=NS)
    nidx = NCHUNK if d_feat <= 64 else NCHUNK // 2
    out_type = [jax.ShapeDtypeStruct((NC, N_PAD, d_feat), jnp.float32)]
    scratch = [
        pltpu.VMEM_SHARED((N_PAD, d_feat), jnp.float32),
        pltpu.VMEM((nidx, CHUNK), jnp.int32),
        pltpu.VMEM((nidx, CHUNK), jnp.int32),
        pltpu.VMEM((CHUNK, d_feat), jnp.float32),
        pltpu.VMEM((CHUNK, d_feat), jnp.float32),
        pltpu.SemaphoreType.DMA,
        pltpu.SemaphoreType.DMA,
    ]
    if with_deg:
        out_type.append(jax.ShapeDtypeStruct((NC, N_PAD, 16), jnp.float32))
        scratch += [
            pltpu.VMEM_SHARED((N_PAD, 16), jnp.float32),
            pltpu.VMEM((CHUNK, 16), jnp.float32),
        ]

    def body(table, src, dst, z_feat, *rest):
        if with_deg:
            (z_deg, acc_out, deg_out, acc_sh, srcv, dstv, rows0, rows1,
             sem0, sem1, deg_sh, onesv) = rest
        else:
            (acc_out, acc_sh, srcv, dstv, rows0, rows1, sem0, sem1) = rest
        cid = lax.axis_index("c")
        sid = lax.axis_index("s")
        wid = sid * NC + cid
        r0 = sid * RPT

        pltpu.sync_copy(z_feat.at[pl.ds(r0, RPT)], acc_sh.at[pl.ds(r0, RPT)])
        if with_deg:
            pltpu.sync_copy(z_deg.at[pl.ds(r0, RPT)], deg_sh.at[pl.ds(r0, RPT)])
            one = jnp.full((16,), 1.0, dtype=jnp.float32)
            for i in range(CHUNK):
                onesv[i] = one
        plsc.subcore_barrier()

        rows = (rows0, rows1)
        sems = (sem0, sem1)

        def gather(k, b):
            return pltpu.async_copy(table.at[srcv.at[k]], rows[b], sems[b])

        def scat(k, b):
            pltpu.sync_copy(rows[b], acc_sh.at[dstv.at[k]], add=True)
            if with_deg:
                pltpu.sync_copy(onesv, deg_sh.at[dstv.at[k]], add=True)

        for h in range(NCHUNK // nidx):
            pltpu.sync_copy(src.at[wid, pl.ds(h * nidx, nidx)], srcv)
            pltpu.sync_copy(dst.at[wid, pl.ds(h * nidx, nidx)], dstv)
            gather(0, 0)

            def pair(p, _):
                k0 = 2 * p
                pltpu.make_async_copy(table.at[srcv.at[k0]], rows0,
                                      sem0).wait()
                gather(k0 + 1, 1)
                scat(k0, 0)
                pltpu.make_async_copy(table.at[srcv.at[k0]], rows1,
                                      sem1).wait()

                @pl.when(p + 1 < nidx // 2)
                def _():
                    gather(k0 + 2, 0)

                scat(k0 + 1, 1)
                return 0

            lax.fori_loop(0, nidx // 2, pair, 0)
        plsc.subcore_barrier()

        pltpu.sync_copy(acc_sh.at[pl.ds(r0, RPT)],
                        acc_out.at[cid, pl.ds(r0, RPT)])
        if with_deg:
            pltpu.sync_copy(deg_sh.at[pl.ds(r0, RPT)],
                            deg_out.at[cid, pl.ds(r0, RPT)])

    return pl.kernel(body, out_type=out_type, mesh=mesh,
                     scratch_types=scratch,
                     compiler_params=pltpu.CompilerParams(
                         use_tc_tiling_on_sc=False),
                     name=f"sage_seg_sum_d{d_feat}")


_seg_sum_l0 = _seg_sum_kernel(D_IN, with_deg=True)
_seg_sum_l1 = _seg_sum_kernel(D_OUT, with_deg=False)

_BLK = 400


def _tc_layer_body(x_ref, accp_ref, accd_ref, ws0_ref, wn0_ref, b0_ref,
                   ws1_ref, wn1_ref, b1_ref, p1_ref, s1e_ref):
    sum_p = accp_ref[0] + accp_ref[1]
    deg = accd_ref[0, :, 0:1] + accd_ref[1, :, 0:1]
    d = jnp.maximum(deg, 1.0)
    agg0 = sum_p / d
    x = x_ref[...]
    h = x @ ws0_ref[...] + agg0 @ wn0_ref[...] + b0_ref[...]
    h = jnp.maximum(h, 0.0)
    p1_ref[...] = jnp.dot(h, wn1_ref[...],
                          preferred_element_type=jnp.float32)
    s1 = jnp.dot(h, ws1_ref[...],
                 preferred_element_type=jnp.float32) + b1_ref[...]
    s1e_ref[...] = jnp.concatenate(
        [s1, jnp.broadcast_to(deg, (s1.shape[0], D_OUT))], axis=1)


def _tc_final_body(s1e_ref, accq_ref, out_ref):
    deg = jnp.maximum(s1e_ref[:, D_OUT:D_OUT + 1], 1.0)
    agg1 = (accq_ref[0] + accq_ref[1]) / deg
    out_ref[...] = s1e_ref[:, :D_OUT] + agg1


def kernel(x, edge_index, W_self0, W_neigh0, b0, W_self1, W_neigh1, b1):
    src = edge_index[0].reshape(NW, NCHUNK, CHUNK)
    dst = edge_index[1].reshape(NW, NCHUNK, CHUNK)
    z128 = jnp.zeros((N_PAD, D_IN), jnp.float32)
    z64 = jnp.zeros((N_PAD, D_OUT), jnp.float32)
    z16 = jnp.zeros((N_PAD, 16), jnp.float32)

    accp, accd = _seg_sum_l0(x, src, dst, z128, z16)

    nblk = N_NODES // _BLK
    p1, s1e = pl.pallas_call(
        _tc_layer_body,
        grid=(nblk,),
        in_specs=[
            pl.BlockSpec((_BLK, D_IN), lambda i: (i, 0)),
            pl.BlockSpec((NC, _BLK, D_IN), lambda i: (0, i, 0)),
            pl.BlockSpec((NC, _BLK, 16), lambda i: (0, i, 0)),
            pl.BlockSpec((D_IN, D_HID), lambda i: (0, 0)),
            pl.BlockSpec((D_IN, D_HID), lambda i: (0, 0)),
            pl.BlockSpec((1, D_HID), lambda i: (0, 0)),
            pl.BlockSpec((D_HID, D_OUT), lambda i: (0, 0)),
            pl.BlockSpec((D_HID, D_OUT), lambda i: (0, 0)),
            pl.BlockSpec((1, D_OUT), lambda i: (0, 0)),
        ],
        out_specs=[
            pl.BlockSpec((_BLK, D_OUT), lambda i: (i, 0)),
            pl.BlockSpec((_BLK, 2 * D_OUT), lambda i: (i, 0)),
        ],
        out_shape=[
            jax.ShapeDtypeStruct((N_NODES, D_OUT), jnp.float32),
            jax.ShapeDtypeStruct((N_NODES, 2 * D_OUT), jnp.float32),
        ],
        name="sage_tc_layer",
    )(x, accp, accd, W_self0, W_neigh0, b0.reshape(1, D_HID),
      W_self1, W_neigh1, b1.reshape(1, D_OUT))

    (accq,) = _seg_sum_l1(p1, src, dst, z64)

    out = pl.pallas_call(
        _tc_final_body,
        grid=(nblk,),
        in_specs=[
            pl.BlockSpec((_BLK, 2 * D_OUT), lambda i: (i, 0)),
            pl.BlockSpec((NC, _BLK, D_OUT), lambda i: (0, i, 0)),
        ],
        out_specs=pl.BlockSpec((_BLK, D_OUT), lambda i: (i, 0)),
        out_shape=jax.ShapeDtypeStruct((N_NODES, D_OUT), jnp.float32),
        name="sage_tc_final",
    )(s1e, accq)
    return out


# speedup vs baseline: 11.2583x; 1.0555x over previous
import functools

import jax
import jax.numpy as jnp
from jax import lax
from jax.experimental import pallas as pl
from jax.experimental.pallas import tpu as pltpu
from jax.experimental.pallas import tpu_sc as plsc

N_NODES = 10000
N_EDGES = 320000
D_IN = 128
D_HID = 128
D_OUT = 64

NC = 2
NS = 16
NW = NC * NS
RPT = N_NODES // NS
EPW = N_EDGES // NW
CHUNK = 125
NCHUNK = EPW // CHUNK


def _seg_sum_kernel(d_feat, with_deg):
    mesh = plsc.VectorSubcoreMesh(core_axis_name="c", subcore_axis_name="s",
                                  num_cores=NC, num_subcores=NS)
    nidx = NCHUNK if d_feat <= 64 else NCHUNK // 4
    out_type = [jax.ShapeDtypeStruct((NC, N_NODES, d_feat), jnp.float32)]
    scratch = [
        pltpu.VMEM_SHARED((N_NODES, d_feat), jnp.float32),
        pltpu.VMEM((nidx, CHUNK), jnp.int32),
        pltpu.VMEM((nidx, CHUNK), jnp.int32),
        pltpu.VMEM((CHUNK, d_feat), jnp.float32),
        pltpu.VMEM((CHUNK, d_feat), jnp.float32),
        pltpu.SemaphoreType.DMA,
        pltpu.SemaphoreType.DMA,
        pltpu.SemaphoreType.DMA,
        pltpu.SemaphoreType.DMA,
    ]
    if with_deg:
        out_type.append(jax.ShapeDtypeStruct((NC, N_NODES, 16), jnp.float32))
        scratch += [
            pltpu.VMEM_SHARED((N_NODES, 16), jnp.float32),
            pltpu.VMEM((CHUNK, 16), jnp.float32),
        ]

    def body(table, src, dst, z_feat, *rest):
        if with_deg:
            (z_deg, acc_out, deg_out, acc_sh, srcv, dstv, rows0, rows1,
             sem0, sem1, ssem0, ssem1, deg_sh, onesv) = rest
        else:
            (acc_out, acc_sh, srcv, dstv, rows0, rows1, sem0, sem1,
             ssem0, ssem1) = rest
        cid = lax.axis_index("c")
        sid = lax.axis_index("s")
        wid = sid * NC + cid
        r0 = sid * RPT

        pltpu.sync_copy(z_feat.at[pl.ds(r0, RPT)], acc_sh.at[pl.ds(r0, RPT)])
        if with_deg:
            pltpu.sync_copy(z_deg.at[pl.ds(r0, RPT)], deg_sh.at[pl.ds(r0, RPT)])
            one = jnp.full((16,), 1.0, dtype=jnp.float32)
            for i in range(CHUNK):
                onesv[i] = one
        plsc.subcore_barrier()

        rows = (rows0, rows1)
        sems = (sem0, sem1)
        ssems = (ssem0, ssem1)

        def gather(k, b):
            pltpu.async_copy(table.at[srcv.at[k]], rows[b], sems[b])

        def gather_wait(b):
            pltpu.make_async_copy(table.at[srcv.at[0]], rows[b],
                                  sems[b]).wait()

        def scat(k, b):
            pltpu.async_copy(rows[b], acc_sh.at[dstv.at[k]], ssems[b],
                             add=True)
            if with_deg:
                pltpu.async_copy(onesv, deg_sh.at[dstv.at[k]], ssems[b],
                                 add=True)

        def scat_wait(b):
            pltpu.make_async_copy(rows[b], acc_sh.at[dstv.at[0]],
                                  ssems[b]).wait()
            if with_deg:
                pltpu.make_async_copy(onesv, deg_sh.at[dstv.at[0]],
                                      ssems[b]).wait()

        for h in range(NCHUNK // nidx):
            pltpu.sync_copy(src.at[wid, pl.ds(h * nidx, nidx)], srcv)
            pltpu.sync_copy(dst.at[wid, pl.ds(h * nidx, nidx)], dstv)
            gather(0, 0)

            def pair(p, _):
                k0 = 2 * p
                gather_wait(0)

                @pl.when(p > 0)
                def _():
                    scat_wait(1)
                gather(k0 + 1, 1)
                scat(k0, 0)
                gather_wait(1)
                scat_wait(0)

                @pl.when(p + 1 < nidx // 2)
                def _():
                    gather(k0 + 2, 0)
                scat(k0 + 1, 1)
                return 0

            lax.fori_loop(0, nidx // 2, pair, 0)
            scat_wait(1)
        plsc.subcore_barrier()

        pltpu.sync_copy(acc_sh.at[pl.ds(r0, RPT)],
                        acc_out.at[cid, pl.ds(r0, RPT)])
        if with_deg:
            pltpu.sync_copy(deg_sh.at[pl.ds(r0, RPT)],
                            deg_out.at[cid, pl.ds(r0, RPT)])

    return pl.kernel(body, out_type=out_type, mesh=mesh,
                     scratch_types=scratch,
                     compiler_params=pltpu.CompilerParams(
                         use_tc_tiling_on_sc=False),
                     name=f"sage_seg_sum_d{d_feat}")


_seg_sum_l0 = _seg_sum_kernel(D_IN, with_deg=True)
_seg_sum_l1 = _seg_sum_kernel(D_OUT, with_deg=False)

_BLK = 400


def _tc_layer_body(x_ref, accp_ref, accd_ref, ws0_ref, wn0_ref, b0_ref,
                   ws1_ref, wn1_ref, b1_ref, p1_ref, s1e_ref):
    sum_p = accp_ref[0] + accp_ref[1]
    deg = accd_ref[0, :, 0:1] + accd_ref[1, :, 0:1]
    d = jnp.maximum(deg, 1.0)
    agg0 = sum_p / d
    x = x_ref[...]
    h = x @ ws0_ref[...] + agg0 @ wn0_ref[...] + b0_ref[...]
    h = jnp.maximum(h, 0.0)
    p1_ref[...] = jnp.dot(h, wn1_ref[...],
                          preferred_element_type=jnp.float32)
    s1 = jnp.dot(h, ws1_ref[...],
                 preferred_element_type=jnp.float32) + b1_ref[...]
    s1e_ref[...] = jnp.concatenate(
        [s1, jnp.broadcast_to(deg, (s1.shape[0], D_OUT))], axis=1)


def _tc_final_body(s1e_ref, accq_ref, out_ref):
    deg = jnp.maximum(s1e_ref[:, D_OUT:D_OUT + 1], 1.0)
    agg1 = (accq_ref[0] + accq_ref[1]) / deg
    out_ref[...] = s1e_ref[:, :D_OUT] + agg1


def kernel(x, edge_index, W_self0, W_neigh0, b0, W_self1, W_neigh1, b1):
    src = edge_index[0].reshape(NW, NCHUNK, CHUNK)
    dst = edge_index[1].reshape(NW, NCHUNK, CHUNK)
    z128 = jnp.zeros((N_NODES, D_IN), jnp.float32)
    z64 = jnp.zeros((N_NODES, D_OUT), jnp.float32)
    z16 = jnp.zeros((N_NODES, 16), jnp.float32)

    accp, accd = _seg_sum_l0(x, src, dst, z128, z16)

    nblk = N_NODES // _BLK
    p1, s1e = pl.pallas_call(
        _tc_layer_body,
        grid=(nblk,),
        in_specs=[
            pl.BlockSpec((_BLK, D_IN), lambda i: (i, 0)),
            pl.BlockSpec((NC, _BLK, D_IN), lambda i: (0, i, 0)),
            pl.BlockSpec((NC, _BLK, 16), lambda i: (0, i, 0)),
            pl.BlockSpec((D_IN, D_HID), lambda i: (0, 0)),
            pl.BlockSpec((D_IN, D_HID), lambda i: (0, 0)),
            pl.BlockSpec((1, D_HID), lambda i: (0, 0)),
            pl.BlockSpec((D_HID, D_OUT), lambda i: (0, 0)),
            pl.BlockSpec((D_HID, D_OUT), lambda i: (0, 0)),
            pl.BlockSpec((1, D_OUT), lambda i: (0, 0)),
        ],
        out_specs=[
            pl.BlockSpec((_BLK, D_OUT), lambda i: (i, 0)),
            pl.BlockSpec((_BLK, 2 * D_OUT), lambda i: (i, 0)),
        ],
        out_shape=[
            jax.ShapeDtypeStruct((N_NODES, D_OUT), jnp.float32),
            jax.ShapeDtypeStruct((N_NODES, 2 * D_OUT), jnp.float32),
        ],
        name="sage_tc_layer",
    )(x, accp, accd, W_self0, W_neigh0, b0.reshape(1, D_HID),
      W_self1, W_neigh1, b1.reshape(1, D_OUT))

    (accq,) = _seg_sum_l1(p1, src, dst, z64)

    out = pl.pallas_call(
        _tc_final_body,
        grid=(nblk,),
        in_specs=[
            pl.BlockSpec((_BLK, 2 * D_OUT), lambda i: (i, 0)),
            pl.BlockSpec((NC, _BLK, D_OUT), lambda i: (0, i, 0)),
        ],
        out_specs=pl.BlockSpec((_BLK, D_OUT), lambda i: (i, 0)),
        out_shape=jax.ShapeDtypeStruct((N_NODES, D_OUT), jnp.float32),
        name="sage_tc_final",
    )(s1e, accq)
    return out


# speedup vs baseline: 12.7425x; 1.1318x over previous
import jax
import jax.numpy as jnp
from jax import lax
from jax.experimental import pallas as pl
from jax.experimental.pallas import tpu as pltpu
from jax.experimental.pallas import tpu_sc as plsc

N_NODES = 10000
N_EDGES = 320000
D_IN = 128
D_HID = 128
D_OUT = 64

NC = 2
NS = 16
NW = NC * NS
N_ACC = 10016
RPT = N_ACC // NS
CHUNK = 128
NCHUNK = 80
EPW = NCHUNK * CHUNK
E_PAD = NW * EPW
N_FAKE = E_PAD - N_EDGES


def _seg_sum_kernel(d_feat, with_deg):
    mesh = plsc.VectorSubcoreMesh(core_axis_name="c", subcore_axis_name="s",
                                  num_cores=NC, num_subcores=NS)
    nidx = NCHUNK if d_feat <= 64 else NCHUNK // 4
    out_type = [jax.ShapeDtypeStruct((NC, N_ACC, d_feat), jnp.float32)]
    scratch = [
        pltpu.VMEM_SHARED((N_ACC, d_feat), jnp.float32),
        pltpu.VMEM((nidx, CHUNK), jnp.int32),
        pltpu.VMEM((nidx, CHUNK), jnp.int32),
        pltpu.VMEM((CHUNK, d_feat), jnp.float32),
        pltpu.VMEM((CHUNK, d_feat), jnp.float32),
        pltpu.SemaphoreType.DMA,
        pltpu.SemaphoreType.DMA,
        pltpu.SemaphoreType.DMA,
        pltpu.SemaphoreType.DMA,
        pltpu.VMEM((CHUNK, 16), jnp.float32),
    ]
    if with_deg:
        out_type.append(jax.ShapeDtypeStruct((NC, N_ACC, 16), jnp.float32))
        scratch.append(pltpu.VMEM_SHARED((N_ACC, 16), jnp.float32))

    def body(table, edges, *rest):
        if with_deg:
            (acc_out, deg_out, acc_sh, srcv, dstv, rows0, rows1,
             sem0, sem1, ssem0, ssem1, onesv, deg_sh) = rest
        else:
            (acc_out, acc_sh, srcv, dstv, rows0, rows1, sem0, sem1,
             ssem0, ssem1, onesv) = rest
        cid = lax.axis_index("c")
        sid = lax.axis_index("s")
        wid = sid * NC + cid
        r0 = sid * RPT

        zvec = jnp.zeros((16,), jnp.float32)

        def zrow(i, _):
            for j in range(d_feat // 16):
                rows0[i, pl.ds(j * 16, 16)] = zvec
            return 0

        lax.fori_loop(0, CHUNK, zrow, 0)
        nfull = RPT // CHUNK
        for c in range(nfull):
            pltpu.sync_copy(rows0, acc_sh.at[pl.ds(r0 + c * CHUNK, CHUNK)])
        rem = RPT - nfull * CHUNK
        if rem:
            pltpu.sync_copy(rows0.at[pl.ds(0, rem)],
                            acc_sh.at[pl.ds(r0 + nfull * CHUNK, rem)])
        if with_deg:
            def z16row(i, _):
                onesv[i, pl.ds(0, 16)] = zvec
                return 0

            lax.fori_loop(0, CHUNK, z16row, 0)
            for c in range(nfull):
                pltpu.sync_copy(onesv,
                                deg_sh.at[pl.ds(r0 + c * CHUNK, CHUNK)])
            if rem:
                pltpu.sync_copy(onesv.at[pl.ds(0, rem)],
                                deg_sh.at[pl.ds(r0 + nfull * CHUNK, rem)])
            one = jnp.full((16,), 1.0, dtype=jnp.float32)

            def orow(i, _):
                onesv[i, pl.ds(0, 16)] = one
                return 0

            lax.fori_loop(0, CHUNK, orow, 0)
        plsc.subcore_barrier()

        rows = (rows0, rows1)
        sems = (sem0, sem1)
        ssems = (ssem0, ssem1)

        def gather(k, b):
            pltpu.async_copy(table.at[srcv.at[k]], rows[b], sems[b])

        def gather_wait(b):
            pltpu.make_async_copy(table.at[srcv.at[0]], rows[b],
                                  sems[b]).wait()

        def scat(k, b):
            pltpu.async_copy(rows[b], acc_sh.at[dstv.at[k]], ssems[b],
                             add=True)
            if with_deg:
                pltpu.async_copy(onesv, deg_sh.at[dstv.at[k]], ssems[b],
                                 add=True)

        def scat_wait(b):
            pltpu.make_async_copy(rows[b], acc_sh.at[dstv.at[0]],
                                  ssems[b]).wait()
            if with_deg:
                pltpu.make_async_copy(onesv, deg_sh.at[dstv.at[0]],
                                      ssems[b]).wait()

        for h in range(NCHUNK // nidx):
            pltpu.sync_copy(edges.at[0, wid, pl.ds(h * nidx, nidx)], srcv)
            pltpu.sync_copy(edges.at[1, wid, pl.ds(h * nidx, nidx)], dstv)
            gather(0, 0)

            def pair(p, _):
                k0 = 2 * p
                gather_wait(0)

                @pl.when(p > 0)
                def _():
                    scat_wait(1)
                gather(k0 + 1, 1)
                scat(k0, 0)
                gather_wait(1)
                scat_wait(0)

                @pl.when(p + 1 < nidx // 2)
                def _():
                    gather(k0 + 2, 0)
                scat(k0 + 1, 1)
                return 0

            lax.fori_loop(0, nidx // 2, pair, 0)
            scat_wait(1)
        plsc.subcore_barrier()

        pltpu.sync_copy(acc_sh.at[pl.ds(r0, RPT)],
                        acc_out.at[cid, pl.ds(r0, RPT)])
        if with_deg:
            pltpu.sync_copy(deg_sh.at[pl.ds(r0, RPT)],
                            deg_out.at[cid, pl.ds(r0, RPT)])

    return pl.kernel(body, out_type=out_type, mesh=mesh,
                     scratch_types=scratch,
                     compiler_params=pltpu.CompilerParams(
                         use_tc_tiling_on_sc=False),
                     name=f"sage_seg_sum_d{d_feat}")


_seg_sum_l0 = _seg_sum_kernel(D_IN, with_deg=True)
_seg_sum_l1 = _seg_sum_kernel(D_OUT, with_deg=False)

_BLK = 2000


def _tc_layer_body(x_ref, accp_ref, accd_ref, ws0_ref, wn0_ref, b0_ref,
                   ws1_ref, wn1_ref, b1_ref, p1_ref, s1e_ref):
    sum_p = accp_ref[0] + accp_ref[1]
    deg = accd_ref[0, :, 0:1] + accd_ref[1, :, 0:1]
    d = jnp.maximum(deg, 1.0)
    agg0 = sum_p / d
    x = x_ref[...]
    h = x @ ws0_ref[...] + agg0 @ wn0_ref[...] + b0_ref[...]
    h = jnp.maximum(h, 0.0)
    p1_ref[...] = jnp.dot(h, wn1_ref[...],
                          preferred_element_type=jnp.float32)
    s1 = jnp.dot(h, ws1_ref[...],
                 preferred_element_type=jnp.float32) + b1_ref[...]
    s1e_ref[...] = jnp.concatenate(
        [s1, jnp.broadcast_to(deg, (s1.shape[0], D_OUT))], axis=1)


def _tc_final_body(s1e_ref, accq_ref, out_ref):
    deg = jnp.maximum(s1e_ref[:, D_OUT:D_OUT + 1], 1.0)
    agg1 = (accq_ref[0] + accq_ref[1]) / deg
    out_ref[...] = s1e_ref[:, :D_OUT] + agg1


def kernel(x, edge_index, W_self0, W_neigh0, b0, W_self1, W_neigh1, b1):
    i = jnp.arange(N_FAKE, dtype=jnp.int32)
    fake = jnp.stack([(i * 131) % N_NODES, N_NODES + (i % 16)])
    edges = jnp.concatenate([edge_index, fake], axis=1)
    edges = edges.reshape(2, NW, NCHUNK, CHUNK)

    accp, accd = _seg_sum_l0(x, edges)

    nblk = N_NODES // _BLK
    p1, s1e = pl.pallas_call(
        _tc_layer_body,
        grid=(nblk,),
        in_specs=[
            pl.BlockSpec((_BLK, D_IN), lambda i: (i, 0)),
            pl.BlockSpec((NC, _BLK, D_IN), lambda i: (0, i, 0)),
            pl.BlockSpec((NC, _BLK, 16), lambda i: (0, i, 0)),
            pl.BlockSpec((D_IN, D_HID), lambda i: (0, 0)),
            pl.BlockSpec((D_IN, D_HID), lambda i: (0, 0)),
            pl.BlockSpec((1, D_HID), lambda i: (0, 0)),
            pl.BlockSpec((D_HID, D_OUT), lambda i: (0, 0)),
            pl.BlockSpec((D_HID, D_OUT), lambda i: (0, 0)),
            pl.BlockSpec((1, D_OUT), lambda i: (0, 0)),
        ],
        out_specs=[
            pl.BlockSpec((_BLK, D_OUT), lambda i: (i, 0)),
            pl.BlockSpec((_BLK, 2 * D_OUT), lambda i: (i, 0)),
        ],
        out_shape=[
            jax.ShapeDtypeStruct((N_NODES, D_OUT), jnp.float32),
            jax.ShapeDtypeStruct((N_NODES, 2 * D_OUT), jnp.float32),
        ],
        name="sage_tc_layer",
    )(x, accp, accd, W_self0, W_neigh0, b0.reshape(1, D_HID),
      W_self1, W_neigh1, b1.reshape(1, D_OUT))

    (accq,) = _seg_sum_l1(p1, edges)

    out = pl.pallas_call(
        _tc_final_body,
        grid=(nblk,),
        in_specs=[
            pl.BlockSpec((_BLK, 2 * D_OUT), lambda i: (i, 0)),
            pl.BlockSpec((NC, _BLK, D_OUT), lambda i: (0, i, 0)),
        ],
        out_specs=pl.BlockSpec((_BLK, D_OUT), lambda i: (i, 0)),
        out_shape=jax.ShapeDtypeStruct((N_NODES, D_OUT), jnp.float32),
        name="sage_tc_final",
    )(s1e, accq)
    return out


# speedup vs baseline: 13.9024x; 1.0910x over previous
import jax
import jax.numpy as jnp
from jax import lax
from jax.experimental import pallas as pl
from jax.experimental.pallas import tpu as pltpu
from jax.experimental.pallas import tpu_sc as plsc

N_NODES = 10000
N_EDGES = 320000
D_IN = 128
D_HID = 128
D_OUT = 64

NC = 2
NS = 16
NW = NC * NS
N_ACC = 10016
RPT = N_ACC // NS
CHUNK = 128
NCHUNK = 80
EPW = NCHUNK * CHUNK
E_PAD = NW * EPW
N_FAKE = E_PAD - N_EDGES


def _seg_sum_kernel(d_feat, with_deg):
    mesh = plsc.VectorSubcoreMesh(core_axis_name="c", subcore_axis_name="s",
                                  num_cores=NC, num_subcores=NS)
    nidx = NCHUNK if d_feat <= 64 else NCHUNK // 4
    nbuf = 2 if with_deg else 4
    out_type = [jax.ShapeDtypeStruct((NC, N_ACC, d_feat), jnp.float32)]
    scratch = [
        pltpu.VMEM_SHARED((N_ACC, d_feat), jnp.float32),
        pltpu.VMEM((nidx, CHUNK), jnp.int32),
        pltpu.VMEM((nidx, CHUNK), jnp.int32),
    ]
    scratch += [pltpu.VMEM((CHUNK, d_feat), jnp.float32)
                for _ in range(nbuf)]
    scratch += [pltpu.SemaphoreType.DMA for _ in range(nbuf)]
    scratch += [pltpu.SemaphoreType.DMA for _ in range(nbuf)]
    scratch += [pltpu.VMEM((CHUNK, 16), jnp.float32)]
    if with_deg:
        out_type.append(jax.ShapeDtypeStruct((NC, N_ACC, 16), jnp.float32))
        scratch.append(pltpu.VMEM_SHARED((N_ACC, 16), jnp.float32))

    def body(table, edges, *rest):
        if with_deg:
            (acc_out, deg_out, acc_sh, srcv, dstv, *rs) = rest
            deg_sh = rest[-1]
        else:
            (acc_out, acc_sh, srcv, dstv, *rs) = rest
        rows = tuple(rs[:nbuf])
        sems = tuple(rs[nbuf:2 * nbuf])
        ssems = tuple(rs[2 * nbuf:3 * nbuf])
        onesv = rs[3 * nbuf]
        rows0 = rows[0]
        cid = lax.axis_index("c")
        sid = lax.axis_index("s")
        wid = sid * NC + cid
        r0 = sid * RPT

        zvec = jnp.zeros((16,), jnp.float32)

        def zrow(i, _):
            for j in range(d_feat // 16):
                rows0[i, pl.ds(j * 16, 16)] = zvec
            return 0

        lax.fori_loop(0, CHUNK, zrow, 0)
        nfull = RPT // CHUNK
        for c in range(nfull):
            pltpu.sync_copy(rows0, acc_sh.at[pl.ds(r0 + c * CHUNK, CHUNK)])
        rem = RPT - nfull * CHUNK
        if rem:
            pltpu.sync_copy(rows0.at[pl.ds(0, rem)],
                            acc_sh.at[pl.ds(r0 + nfull * CHUNK, rem)])
        if with_deg:
            def z16row(i, _):
                onesv[i, pl.ds(0, 16)] = zvec
                return 0

            lax.fori_loop(0, CHUNK, z16row, 0)
            for c in range(nfull):
                pltpu.sync_copy(onesv,
                                deg_sh.at[pl.ds(r0 + c * CHUNK, CHUNK)])
            if rem:
                pltpu.sync_copy(onesv.at[pl.ds(0, rem)],
                                deg_sh.at[pl.ds(r0 + nfull * CHUNK, rem)])
            one = jnp.full((16,), 1.0, dtype=jnp.float32)

            def orow(i, _):
                onesv[i, pl.ds(0, 16)] = one
                return 0

            lax.fori_loop(0, CHUNK, orow, 0)
        plsc.subcore_barrier()

        def gather(k, b):
            pltpu.async_copy(table.at[srcv.at[k]], rows[b], sems[b])

        def gather_wait(b):
            pltpu.make_async_copy(table.at[srcv.at[0]], rows[b],
                                  sems[b]).wait()

        def scat(k, b):
            pltpu.async_copy(rows[b], acc_sh.at[dstv.at[k]], ssems[b],
                             add=True)
            if with_deg:
                pltpu.async_copy(onesv, deg_sh.at[dstv.at[k]], ssems[b],
                                 add=True)

        def scat_wait(b):
            pltpu.make_async_copy(rows[b], acc_sh.at[dstv.at[0]],
                                  ssems[b]).wait()
            if with_deg:
                pltpu.make_async_copy(onesv, deg_sh.at[dstv.at[0]],
                                      ssems[b]).wait()

        for h in range(NCHUNK // nidx):
            pltpu.sync_copy(edges.at[0, wid, pl.ds(h * nidx, nidx)], srcv)
            pltpu.sync_copy(edges.at[1, wid, pl.ds(h * nidx, nidx)], dstv)
            if nbuf == 2:
                gather(0, 0)

                def pair(p, _):
                    k0 = 2 * p
                    gather_wait(0)

                    @pl.when(p > 0)
                    def _():
                        scat_wait(1)
                    gather(k0 + 1, 1)
                    scat(k0, 0)
                    gather_wait(1)
                    scat_wait(0)

                    @pl.when(p + 1 < nidx // 2)
                    def _():
                        gather(k0 + 2, 0)
                    scat(k0 + 1, 1)
                    return 0

                lax.fori_loop(0, nidx // 2, pair, 0)
                scat_wait(1)
            else:
                gather(0, 0)
                gather(1, 1)

                def quad(q, _):
                    for j in range(4):
                        k = 4 * q + j
                        b = j
                        gather_wait(b)
                        scat(k, b)
                        bn = (j + 2) % 4

                        @pl.when(k >= 2)
                        def _():
                            scat_wait(bn)

                        @pl.when(k + 2 < nidx)
                        def _():
                            gather(k + 2, bn)
                    return 0

                lax.fori_loop(0, nidx // 4, quad, 0)
                scat_wait(2)
                scat_wait(3)
        plsc.subcore_barrier()

        pltpu.sync_copy(acc_sh.at[pl.ds(r0, RPT)],
                        acc_out.at[cid, pl.ds(r0, RPT)])
        if with_deg:
            pltpu.sync_copy(deg_sh.at[pl.ds(r0, RPT)],
                            deg_out.at[cid, pl.ds(r0, RPT)])

    return pl.kernel(body, out_type=out_type, mesh=mesh,
                     scratch_types=scratch,
                     compiler_params=pltpu.CompilerParams(
                         use_tc_tiling_on_sc=False),
                     name=f"sage_seg_sum_d{d_feat}")


_seg_sum_l0 = _seg_sum_kernel(D_IN, with_deg=True)
_seg_sum_l1 = _seg_sum_kernel(D_OUT, with_deg=False)

_BLK = 2000


def _tc_layer_body(x_ref, accp_ref, accd_ref, ws0_ref, wn0_ref, b0_ref,
                   ws1_ref, wn1_ref, b1_ref, p1_ref, s1e_ref):
    sum_p = accp_ref[0] + accp_ref[1]
    deg = accd_ref[0, :, 0:1] + accd_ref[1, :, 0:1]
    d = jnp.maximum(deg, 1.0)
    agg0 = sum_p / d
    x = x_ref[...]
    h = x @ ws0_ref[...] + agg0 @ wn0_ref[...] + b0_ref[...]
    h = jnp.maximum(h, 0.0)
    p1_ref[...] = jnp.dot(h, wn1_ref[...],
                          preferred_element_type=jnp.float32)
    s1 = jnp.dot(h, ws1_ref[...],
                 preferred_element_type=jnp.float32) + b1_ref[...]
    s1e_ref[...] = jnp.concatenate(
        [s1, jnp.broadcast_to(deg, (s1.shape[0], D_OUT))], axis=1)


def _tc_final_body(s1e_ref, accq_ref, out_ref):
    deg = jnp.maximum(s1e_ref[:, D_OUT:D_OUT + 1], 1.0)
    agg1 = (accq_ref[0] + accq_ref[1]) / deg
    out_ref[...] = s1e_ref[:, :D_OUT] + agg1


def kernel(x, edge_index, W_self0, W_neigh0, b0, W_self1, W_neigh1, b1):
    i = jnp.arange(N_FAKE, dtype=jnp.int32)
    fake = jnp.stack([(i * 131) % N_NODES, N_NODES + (i % 16)])
    edges = jnp.concatenate([edge_index, fake], axis=1)
    edges = edges.reshape(2, NW, NCHUNK, CHUNK)

    accp, accd = _seg_sum_l0(x, edges)

    nblk = N_NODES // _BLK
    p1, s1e = pl.pallas_call(
        _tc_layer_body,
        grid=(nblk,),
        in_specs=[
            pl.BlockSpec((_BLK, D_IN), lambda i: (i, 0)),
            pl.BlockSpec((NC, _BLK, D_IN), lambda i: (0, i, 0)),
            pl.BlockSpec((NC, _BLK, 16), lambda i: (0, i, 0)),
            pl.BlockSpec((D_IN, D_HID), lambda i: (0, 0)),
            pl.BlockSpec((D_IN, D_HID), lambda i: (0, 0)),
            pl.BlockSpec((1, D_HID), lambda i: (0, 0)),
            pl.BlockSpec((D_HID, D_OUT), lambda i: (0, 0)),
            pl.BlockSpec((D_HID, D_OUT), lambda i: (0, 0)),
            pl.BlockSpec((1, D_OUT), lambda i: (0, 0)),
        ],
        out_specs=[
            pl.BlockSpec((_BLK, D_OUT), lambda i: (i, 0)),
            pl.BlockSpec((_BLK, 2 * D_OUT), lambda i: (i, 0)),
        ],
        out_shape=[
            jax.ShapeDtypeStruct((N_NODES, D_OUT), jnp.float32),
            jax.ShapeDtypeStruct((N_NODES, 2 * D_OUT), jnp.float32),
        ],
        name="sage_tc_layer",
    )(x, accp, accd, W_self0, W_neigh0, b0.reshape(1, D_HID),
      W_self1, W_neigh1, b1.reshape(1, D_OUT))

    (accq,) = _seg_sum_l1(p1, edges)

    out = pl.pallas_call(
        _tc_final_body,
        grid=(nblk,),
        in_specs=[
            pl.BlockSpec((_BLK, 2 * D_OUT), lambda i: (i, 0)),
            pl.BlockSpec((NC, _BLK, D_OUT), lambda i: (0, i, 0)),
        ],
        out_specs=pl.BlockSpec((_BLK, D_OUT), lambda i: (i, 0)),
        out_shape=jax.ShapeDtypeStruct((N_NODES, D_OUT), jnp.float32),
        name="sage_tc_final",
    )(s1e, accq)
    return out
